# baseline (speedup 1.0000x reference)
import functools

import jax
import jax.numpy as jnp
from jax import lax
from jax.experimental import pallas as pl
from jax.experimental.pallas import tpu as pltpu
from jax.experimental.pallas import tpu_sc as plsc

N = 320000
B = 4096
D = 128
H = 64

MEAN = 0.7546106515883616
STD = 0.30338715545464656
A_TO_A0 = 1.8897268777743552

NSC = 32
SEG_PER = B // NSC

BLKA = 2048
NBLK = -(-N // BLKA)
NP = NBLK * BLKA
PAD_SEG = 2 * B

CH = 2048

_MASS16 = jnp.array(
    [0.0, 1.00784, 0.0, 0.0, 0.0, 0.0, 12.0107, 14.0067, 15.999, 18.998403,
     0.0, 0.0, 0.0, 0.0, 0.0, 0.0], dtype=jnp.float32)


def _mlp_body(x_ref, w0_ref, b0_ref, w1_ref, b1_ref, batch_ref, q_ref,
              cnt_ref):
    i = pl.program_id(0)
    x = x_ref[...]
    h = lax.dot_general(x, w0_ref[...], (((1,), (1,)), ((), ())),
                        preferred_element_type=jnp.float32)
    h = h + b0_ref[...]
    h = h * jax.nn.sigmoid(h)
    q = lax.dot_general(w1_ref[...], h, (((1,), (1,)), ((), ())),
                        preferred_element_type=jnp.float32)
    q = (q + b1_ref[...]) * STD + MEAN
    q_ref[...] = q.reshape(BLKA)

    bb = batch_ref[...]
    th = lax.broadcasted_iota(jnp.int32, (64, BLKA), 0) * SEG_PER
    cmp = (bb[None, :] < th).astype(jnp.int32)
    partial = jnp.sum(cmp, axis=1, keepdims=True)

    @pl.when(i == 0)
    def _():
        cnt_ref[...] = jnp.zeros_like(cnt_ref)

    cnt_ref[...] += jnp.broadcast_to(partial, (64, 8))


def _mlp_call(x, W0, b0_2d, W1, b1_2d, batchp):
    vec = pl.BlockSpec((BLKA,), lambda i: (i,))
    return pl.pallas_call(
        _mlp_body,
        grid=(NBLK,),
        in_specs=[
            pl.BlockSpec((BLKA, D), lambda i: (i, 0)),
            pl.BlockSpec((H, D), lambda i: (0, 0)),
            pl.BlockSpec((1, H), lambda i: (0, 0)),
            pl.BlockSpec((1, H), lambda i: (0, 0)),
            pl.BlockSpec((1, 1), lambda i: (0, 0)),
            vec,
        ],
        out_specs=[vec, pl.BlockSpec((64, 8), lambda i: (0, 0))],
        out_shape=[
            jax.ShapeDtypeStruct((NP,), jnp.float32),
            jax.ShapeDtypeStruct((64, 8), jnp.int32),
        ],
    )(x, W0, b0_2d, W1, b1_2d, batchp)


def _bound(cnt_vm, j):
    jv = jnp.full((16,), j, jnp.int32)
    z = jnp.zeros((16,), jnp.int32)
    return jnp.max(plsc.load_gather(cnt_vm, [jv, z]))


def _sc_body(batch_hbm, z_hbm, q_hbm, pos_hbm, cnt_hbm, mass_hbm, out_hbm,
             cnt_vm, mass_vm, bufs, acc_vm, cmx_vm, cmy_vm, cmz_vm, mq_vm,
             out_vm, sems):
    c = lax.axis_index("c")
    s = lax.axis_index("s")
    w = s * 2 + c
    segbase = w * SEG_PER

    pltpu.sync_copy(cnt_hbm, cnt_vm)
    pltpu.sync_copy(mass_hbm, mass_vm)
    start = _bound(cnt_vm, w)
    end = _bound(cnt_vm, w + 1)

    lane = lax.iota(jnp.int32, 16)
    zero16i = jnp.zeros((16,), jnp.int32)
    one16i = jnp.full((16,), 1, jnp.int32)
    two16i = jnp.full((16,), 2, jnp.int32)
    zero16f = jnp.zeros((16,), jnp.float32)
    one16f = jnp.ones((16,), jnp.float32)

    for k in range(SEG_PER * 8 // 16):
        acc_vm[pl.ds(k * 16, 16)] = zero16f
    for k in range(SEG_PER // 16):
        out_vm[pl.ds(k * 16, 16)] = zero16f

    t0 = start // CH
    t1 = (end + CH - 1) // CH
    nch = t1 - t0
    npairs = (nch + 1) // 2

    def issue(t, slot):
        base = t * CH
        base_p = jnp.minimum(base, N - CH)
        sem = sems.at[slot]
        bat_vm, z_vm, q_vm, pos_vm = bufs[slot]
        pltpu.async_copy(batch_hbm.at[pl.ds(base, CH)], bat_vm, sem)
        pltpu.async_copy(z_hbm.at[pl.ds(base, CH)], z_vm, sem)
        pltpu.async_copy(q_hbm.at[pl.ds(base, CH)], q_vm, sem)
        pltpu.async_copy(pos_hbm.at[pl.ds(base_p, CH)], pos_vm, sem)

    def drain(slot):
        sem = sems.at[slot]
        bat_vm, z_vm, q_vm, pos_vm = bufs[slot]
        pltpu.make_async_copy(batch_hbm.at[pl.ds(0, CH)], bat_vm, sem).wait()
        pltpu.make_async_copy(z_hbm.at[pl.ds(0, CH)], z_vm, sem).wait()
        pltpu.make_async_copy(q_hbm.at[pl.ds(0, CH)], q_vm, sem).wait()
        pltpu.make_async_copy(pos_hbm.at[pl.ds(0, CH)], pos_vm, sem).wait()

    def load_vregs(base, k, slot):
        bat_vm, z_vm, q_vm, pos_vm = bufs[slot]
        off = k * 16
        delta = base - jnp.minimum(base, N - CH)
        b16 = bat_vm[pl.ds(off, 16)]
        z16 = z_vm[pl.ds(off, 16)]
        q16 = q_vm[pl.ds(off, 16)]
        aidx = base + off + lane
        msk = (aidx >= start) & (aidx < end)
        ridx = delta + off + lane
        px = plsc.load_gather(pos_vm, [ridx, zero16i], mask=msk)
        py = plsc.load_gather(pos_vm, [ridx, one16i], mask=msk)
        pz = plsc.load_gather(pos_vm, [ridx, two16i], mask=msk)
        rel = jnp.clip(b16 - segbase, 0, SEG_PER - 1)
        return z16, q16, px, py, pz, msk, rel

    def pass1_vreg(base, k, slot):
        z16, q16, px, py, pz, msk, rel = load_vregs(base, k, slot)
        m16 = plsc.load_gather(mass_vm, [z16])
        i8 = rel * 8
        plsc.addupdate_scatter(acc_vm, [i8], m16, mask=msk)
        plsc.addupdate_scatter(acc_vm, [i8 + 1], m16 * px, mask=msk)
        plsc.addupdate_scatter(acc_vm, [i8 + 2], m16 * py, mask=msk)
        plsc.addupdate_scatter(acc_vm, [i8 + 3], m16 * pz, mask=msk)
        plsc.addupdate_scatter(acc_vm, [i8 + 4], q16, mask=msk)
        plsc.addupdate_scatter(acc_vm, [i8 + 5], one16f, mask=msk)

    def pass2_vreg(base, k, slot):
        z16, q16, px, py, pz, msk, rel = load_vregs(base, k, slot)
        cmx = plsc.load_gather(cmx_vm, [rel])
        cmy = plsc.load_gather(cmy_vm, [rel])
        cmz = plsc.load_gather(cmz_vm, [rel])
        mq = plsc.load_gather(mq_vm, [rel])
        dx = (px - cmx) * A_TO_A0
        dy = (py - cmy) * A_TO_A0
        dz = (pz - cmz) * A_TO_A0
        r2 = dx * dx + dy * dy + dz * dz
        cloud = jnp.abs(q16 - mq - z16.astype(jnp.float32))
        plsc.addupdate_scatter(out_vm, [rel], cloud * r2, mask=msk)

    def run_pass(vreg_fn):
        @pl.when(nch > 0)
        def _():
            issue(t0, 0)

        def pair_body(j, carry):
            t_a = t0 + 2 * j
            t_b = t_a + 1

            @pl.when(t_b < t1)
            def _():
                issue(t_b, 1)

            drain(0)

            def inner_a(k, c2):
                vreg_fn(t_a * CH, k, 0)
                return c2

            lax.fori_loop(0, CH // 16, inner_a, 0, unroll=4)

            @pl.when(t_b + 1 < t1)
            def _():
                issue(t_b + 1, 0)

            @pl.when(t_b < t1)
            def _():
                drain(1)

                def inner_b(k, c2):
                    vreg_fn(t_b * CH, k, 1)
                    return c2

                lax.fori_loop(0, CH // 16, inner_b, 0, unroll=4)

            return carry

        lax.fori_loop(0, npairs, pair_body, 0)

    run_pass(pass1_vreg)

    for k in range(SEG_PER // 16):
        sidx = (k * 16 + lane) * 8
        sm = plsc.load_gather(acc_vm, [sidx])
        smx = plsc.load_gather(acc_vm, [sidx + 1])
        smy = plsc.load_gather(acc_vm, [sidx + 2])
        smz = plsc.load_gather(acc_vm, [sidx + 3])
        sq = plsc.load_gather(acc_vm, [sidx + 4])
        n = plsc.load_gather(acc_vm, [sidx + 5])
        cmx_vm[pl.ds(k * 16, 16)] = smx / sm
        cmy_vm[pl.ds(k * 16, 16)] = smy / sm
        cmz_vm[pl.ds(k * 16, 16)] = smz / sm
        mq_vm[pl.ds(k * 16, 16)] = sq / n

    run_pass(pass2_vreg)

    pltpu.sync_copy(out_vm, out_hbm.at[pl.ds(segbase, SEG_PER)])


_sc_call = functools.partial(
    pl.kernel,
    out_type=jax.ShapeDtypeStruct((B,), jnp.float32),
    mesh=plsc.VectorSubcoreMesh(core_axis_name="c", subcore_axis_name="s"),
    scratch_types=[
        pltpu.VMEM((64, 8), jnp.int32),
        pltpu.VMEM((16,), jnp.float32),
        [[pltpu.VMEM((CH,), jnp.int32),
          pltpu.VMEM((CH,), jnp.int32),
          pltpu.VMEM((CH,), jnp.float32),
          pltpu.VMEM((CH, 3), jnp.float32)]
         for _ in range(2)],
        pltpu.VMEM((SEG_PER * 8,), jnp.float32),
        pltpu.VMEM((SEG_PER,), jnp.float32),
        pltpu.VMEM((SEG_PER,), jnp.float32),
        pltpu.VMEM((SEG_PER,), jnp.float32),
        pltpu.VMEM((SEG_PER,), jnp.float32),
        pltpu.VMEM((SEG_PER,), jnp.float32),
        pltpu.SemaphoreType.DMA((2,)),
    ],
    compiler_params=pltpu.CompilerParams(needs_layout_passes=False,
                                         use_tc_tiling_on_sc=False),
)(_sc_body)


def kernel(x, pos, Z, batch, W0, b0, W1, b1):
    batchp = jnp.pad(batch.astype(jnp.int32), (0, NP - N),
                     constant_values=PAD_SEG)
    zp = jnp.pad(Z.reshape(N).astype(jnp.int32), (0, NP - N))
    q1, cnt = _mlp_call(x, W0, b0.reshape(1, H), W1, b1.reshape(1, 1), batchp)
    out = _sc_call(batchp, zp, q1, pos, cnt, _MASS16)
    return out.reshape(B, 1)


# speedup vs baseline: 1.2652x; 1.2652x over previous
import functools

import jax
import jax.numpy as jnp
from jax import lax
from jax.experimental import pallas as pl
from jax.experimental.pallas import tpu as pltpu
from jax.experimental.pallas import tpu_sc as plsc

N = 320000
B = 4096
D = 128
H = 64

MEAN = 0.7546106515883616
STD = 0.30338715545464656
A_TO_A0 = 1.8897268777743552

NSC = 32
SEG_PER = B // NSC

BLKA = 2048
NBLK = -(-N // BLKA)
NP = NBLK * BLKA
PAD_SEG = 2 * B

CH = 2048

_MASS16 = jnp.array(
    [0.0, 1.00784, 0.0, 0.0, 0.0, 0.0, 12.0107, 14.0067, 15.999, 18.998403,
     0.0, 0.0, 0.0, 0.0, 0.0, 0.0], dtype=jnp.float32)

_SEL43 = jnp.concatenate(
    [jnp.eye(3, dtype=jnp.float32), jnp.zeros((1, 3), jnp.float32)], axis=0)


def _mlp_body(x_ref, pos_ref, sel_ref, w0_ref, b0_ref, w1_ref, b1_ref,
              batch_ref, q_ref, px_ref, py_ref, pz_ref, cnt_ref):
    i = pl.program_id(0)
    x = x_ref[...]
    h = lax.dot_general(x, w0_ref[...], (((1,), (1,)), ((), ())),
                        preferred_element_type=jnp.float32)
    h = h + b0_ref[...]
    h = h * jax.nn.sigmoid(h)
    q = lax.dot_general(w1_ref[...], h, (((1,), (1,)), ((), ())),
                        preferred_element_type=jnp.float32)
    q = (q + b1_ref[...]) * STD + MEAN
    q_ref[...] = q.reshape(BLKA)

    pxyz = lax.dot_general(sel_ref[...], pos_ref[...], (((1,), (1,)), ((), ())),
                           precision=lax.Precision.HIGHEST,
                           preferred_element_type=jnp.float32)
    px_ref[...] = pxyz[0:1, :].reshape(BLKA)
    py_ref[...] = pxyz[1:2, :].reshape(BLKA)
    pz_ref[...] = pxyz[2:3, :].reshape(BLKA)

    bb = batch_ref[...]
    th = lax.broadcasted_iota(jnp.int32, (64, BLKA), 0) * SEG_PER
    cmp = (bb[None, :] < th).astype(jnp.int32)
    partial = jnp.sum(cmp, axis=1, keepdims=True)

    @pl.when(i == 0)
    def _():
        cnt_ref[...] = jnp.zeros_like(cnt_ref)

    cnt_ref[...] += jnp.broadcast_to(partial, (64, 8))


def _mlp_call(x, pos, W0, b0_2d, W1, b1_2d, batchp):
    vec = pl.BlockSpec((BLKA,), lambda i: (i,))
    vout = jax.ShapeDtypeStruct((NP,), jnp.float32)
    return pl.pallas_call(
        _mlp_body,
        grid=(NBLK,),
        in_specs=[
            pl.BlockSpec((BLKA, D), lambda i: (i, 0)),
            pl.BlockSpec((BLKA, 3), lambda i: (i, 0)),
            pl.BlockSpec((4, 3), lambda i: (0, 0)),
            pl.BlockSpec((H, D), lambda i: (0, 0)),
            pl.BlockSpec((1, H), lambda i: (0, 0)),
            pl.BlockSpec((1, H), lambda i: (0, 0)),
            pl.BlockSpec((1, 1), lambda i: (0, 0)),
            vec,
        ],
        out_specs=[vec, vec, vec, vec, pl.BlockSpec((64, 8), lambda i: (0, 0))],
        out_shape=[vout, vout, vout, vout,
                   jax.ShapeDtypeStruct((64, 8), jnp.int32)],
    )(x, pos, _SEL43, W0, b0_2d, W1, b1_2d, batchp)


def _bound(cnt_vm, j):
    jv = jnp.full((16,), j, jnp.int32)
    z = jnp.zeros((16,), jnp.int32)
    return jnp.max(plsc.load_gather(cnt_vm, [jv, z]))


def _sc_body(batch_hbm, z_hbm, q_hbm, px_hbm, py_hbm, pz_hbm, cnt_hbm,
             mass_hbm, out_hbm,
             cnt_vm, mass_vm, bufs, acc_vm, cmx_vm, cmy_vm, cmz_vm, mq_vm,
             out_vm, sems):
    c = lax.axis_index("c")
    s = lax.axis_index("s")
    w = s * 2 + c
    segbase = w * SEG_PER

    pltpu.sync_copy(cnt_hbm, cnt_vm)
    pltpu.sync_copy(mass_hbm, mass_vm)
    start = _bound(cnt_vm, w)
    end = _bound(cnt_vm, w + 1)

    lane = lax.iota(jnp.int32, 16)
    zero16i = jnp.zeros((16,), jnp.int32)
    one16i = jnp.full((16,), 1, jnp.int32)
    two16i = jnp.full((16,), 2, jnp.int32)
    zero16f = jnp.zeros((16,), jnp.float32)
    one16f = jnp.ones((16,), jnp.float32)

    for k in range(SEG_PER * 8 // 16):
        acc_vm[pl.ds(k * 16, 16)] = zero16f
    for k in range(SEG_PER // 16):
        out_vm[pl.ds(k * 16, 16)] = zero16f

    t0 = start // CH
    t1 = (end + CH - 1) // CH
    nch = t1 - t0
    npairs = (nch + 1) // 2

    srcs = (batch_hbm, z_hbm, q_hbm, px_hbm, py_hbm, pz_hbm)

    def issue(t, slot):
        base = t * CH
        sem = sems.at[slot]
        for src, dst in zip(srcs, bufs[slot]):
            pltpu.async_copy(src.at[pl.ds(base, CH)], dst, sem)

    def drain(slot):
        sem = sems.at[slot]
        for src, dst in zip(srcs, bufs[slot]):
            pltpu.make_async_copy(src.at[pl.ds(0, CH)], dst, sem).wait()

    def load_vregs(base, k, slot):
        bat_vm, z_vm, q_vm, px_vm, py_vm, pz_vm = bufs[slot]
        off = k * 16
        b16 = bat_vm[pl.ds(off, 16)]
        z16 = z_vm[pl.ds(off, 16)]
        q16 = q_vm[pl.ds(off, 16)]
        px = px_vm[pl.ds(off, 16)]
        py = py_vm[pl.ds(off, 16)]
        pz = pz_vm[pl.ds(off, 16)]
        aidx = base + off + lane
        msk = (aidx >= start) & (aidx < end)
        rel = jnp.clip(b16 - segbase, 0, SEG_PER - 1)
        return z16, q16, px, py, pz, msk, rel

    def pass1_vreg(base, k, slot):
        z16, q16, px, py, pz, msk, rel = load_vregs(base, k, slot)
        m16 = plsc.load_gather(mass_vm, [z16])
        i8 = rel * 8
        plsc.addupdate_scatter(acc_vm, [i8], m16, mask=msk)
        plsc.addupdate_scatter(acc_vm, [i8 + 1], m16 * px, mask=msk)
        plsc.addupdate_scatter(acc_vm, [i8 + 2], m16 * py, mask=msk)
        plsc.addupdate_scatter(acc_vm, [i8 + 3], m16 * pz, mask=msk)
        plsc.addupdate_scatter(acc_vm, [i8 + 4], q16, mask=msk)
        plsc.addupdate_scatter(acc_vm, [i8 + 5], one16f, mask=msk)

    def pass2_vreg(base, k, slot):
        z16, q16, px, py, pz, msk, rel = load_vregs(base, k, slot)
        cmx = plsc.load_gather(cmx_vm, [rel])
        cmy = plsc.load_gather(cmy_vm, [rel])
        cmz = plsc.load_gather(cmz_vm, [rel])
        mq = plsc.load_gather(mq_vm, [rel])
        dx = (px - cmx) * A_TO_A0
        dy = (py - cmy) * A_TO_A0
        dz = (pz - cmz) * A_TO_A0
        r2 = dx * dx + dy * dy + dz * dz
        cloud = jnp.abs(q16 - mq - z16.astype(jnp.float32))
        plsc.addupdate_scatter(out_vm, [rel], cloud * r2, mask=msk)

    def run_pass(vreg_fn):
        @pl.when(nch > 0)
        def _():
            issue(t0, 0)

        def pair_body(j, carry):
            t_a = t0 + 2 * j
            t_b = t_a + 1

            @pl.when(t_b < t1)
            def _():
                issue(t_b, 1)

            drain(0)

            def inner_a(k, c2):
                vreg_fn(t_a * CH, k, 0)
                return c2

            lax.fori_loop(0, CH // 16, inner_a, 0, unroll=4)

            @pl.when(t_b + 1 < t1)
            def _():
                issue(t_b + 1, 0)

            @pl.when(t_b < t1)
            def _():
                drain(1)

                def inner_b(k, c2):
                    vreg_fn(t_b * CH, k, 1)
                    return c2

                lax.fori_loop(0, CH // 16, inner_b, 0, unroll=4)

            return carry

        lax.fori_loop(0, npairs, pair_body, 0)

    run_pass(pass1_vreg)

    for k in range(SEG_PER // 16):
        sidx = (k * 16 + lane) * 8
        sm = plsc.load_gather(acc_vm, [sidx])
        smx = plsc.load_gather(acc_vm, [sidx + 1])
        smy = plsc.load_gather(acc_vm, [sidx + 2])
        smz = plsc.load_gather(acc_vm, [sidx + 3])
        sq = plsc.load_gather(acc_vm, [sidx + 4])
        n = plsc.load_gather(acc_vm, [sidx + 5])
        cmx_vm[pl.ds(k * 16, 16)] = smx / sm
        cmy_vm[pl.ds(k * 16, 16)] = smy / sm
        cmz_vm[pl.ds(k * 16, 16)] = smz / sm
        mq_vm[pl.ds(k * 16, 16)] = sq / n

    run_pass(pass2_vreg)

    pltpu.sync_copy(out_vm, out_hbm.at[pl.ds(segbase, SEG_PER)])


_sc_call = functools.partial(
    pl.kernel,
    out_type=jax.ShapeDtypeStruct((B,), jnp.float32),
    mesh=plsc.VectorSubcoreMesh(core_axis_name="c", subcore_axis_name="s"),
    scratch_types=[
        pltpu.VMEM((64, 8), jnp.int32),
        pltpu.VMEM((16,), jnp.float32),
        [[pltpu.VMEM((CH,), jnp.int32),
          pltpu.VMEM((CH,), jnp.int32),
          pltpu.VMEM((CH,), jnp.float32),
          pltpu.VMEM((CH,), jnp.float32),
          pltpu.VMEM((CH,), jnp.float32),
          pltpu.VMEM((CH,), jnp.float32)]
         for _ in range(2)],
        pltpu.VMEM((SEG_PER * 8,), jnp.float32),
        pltpu.VMEM((SEG_PER,), jnp.float32),
        pltpu.VMEM((SEG_PER,), jnp.float32),
        pltpu.VMEM((SEG_PER,), jnp.float32),
        pltpu.VMEM((SEG_PER,), jnp.float32),
        pltpu.VMEM((SEG_PER,), jnp.float32),
        pltpu.SemaphoreType.DMA((2,)),
    ],
    compiler_params=pltpu.CompilerParams(needs_layout_passes=False,
                                         use_tc_tiling_on_sc=False),
)(_sc_body)


def kernel(x, pos, Z, batch, W0, b0, W1, b1):
    batchp = jnp.pad(batch.astype(jnp.int32), (0, NP - N),
                     constant_values=PAD_SEG)
    zp = jnp.pad(Z.reshape(N).astype(jnp.int32), (0, NP - N))
    q1, px1, py1, pz1, cnt = _mlp_call(x, pos, W0, b0.reshape(1, H),
                                       W1, b1.reshape(1, 1), batchp)
    out = _sc_call(batchp, zp, q1, px1, py1, pz1, cnt, _MASS16)
    return out.reshape(B, 1)


# speedup vs baseline: 1.9807x; 1.5655x over previous
import functools

import jax
import jax.numpy as jnp
from jax import lax
from jax.experimental import pallas as pl
from jax.experimental.pallas import tpu as pltpu
from jax.experimental.pallas import tpu_sc as plsc

N = 320000
B = 4096
D = 128
H = 64

MEAN = 0.7546106515883616
STD = 0.30338715545464656
A_TO_A0 = 1.8897268777743552

NSC = 32
SEG_PER = B // NSC

BLKA = 2048
NBLK = -(-N // BLKA)
NP = NBLK * BLKA
PAD_SEG = 2 * B

CH = 2048

_MASS16 = jnp.array(
    [0.0, 1.00784, 0.0, 0.0, 0.0, 0.0, 12.0107, 14.0067, 15.999, 18.998403,
     0.0, 0.0, 0.0, 0.0, 0.0, 0.0], dtype=jnp.float32)

_SEL43 = jnp.concatenate(
    [jnp.eye(3, dtype=jnp.float32), jnp.zeros((1, 3), jnp.float32)], axis=0)


def _mlp_body(x_ref, w0_ref, b0_ref, w1_ref, b1_ref, batch_ref, q_ref,
              cnt_ref):
    i = pl.program_id(0)
    x = x_ref[...]
    h = lax.dot_general(x, w0_ref[...], (((1,), (1,)), ((), ())),
                        preferred_element_type=jnp.float32)
    h = h + b0_ref[...]
    h = h * jax.nn.sigmoid(h)
    q = lax.dot_general(w1_ref[...], h, (((1,), (1,)), ((), ())),
                        preferred_element_type=jnp.float32)
    q = (q + b1_ref[...]) * STD + MEAN
    q_ref[...] = q.reshape(BLKA)

    bb = batch_ref[...]
    th = lax.broadcasted_iota(jnp.int32, (64, BLKA), 0) * SEG_PER
    cmp = (bb[None, :] < th).astype(jnp.int32)
    partial = jnp.sum(cmp, axis=1, keepdims=True)

    @pl.when(i == 0)
    def _():
        cnt_ref[...] = jnp.zeros_like(cnt_ref)

    cnt_ref[...] += jnp.broadcast_to(partial, (64, 8))


def _mlp_call(x, W0, b0_2d, W1, b1_2d, batchp):
    vec = pl.BlockSpec((BLKA,), lambda i: (i,))
    return pl.pallas_call(
        _mlp_body,
        grid=(NBLK,),
        in_specs=[
            pl.BlockSpec((BLKA, D), lambda i: (i, 0)),
            pl.BlockSpec((H, D), lambda i: (0, 0)),
            pl.BlockSpec((1, H), lambda i: (0, 0)),
            pl.BlockSpec((1, H), lambda i: (0, 0)),
            pl.BlockSpec((1, 1), lambda i: (0, 0)),
            vec,
        ],
        out_specs=[vec, pl.BlockSpec((64, 8), lambda i: (0, 0))],
        out_shape=[
            jax.ShapeDtypeStruct((NP,), jnp.float32),
            jax.ShapeDtypeStruct((64, 8), jnp.int32),
        ],
    )(x, W0, b0_2d, W1, b1_2d, batchp)


def _bound(cnt_vm, j):
    jv = jnp.full((16,), j, jnp.int32)
    z = jnp.zeros((16,), jnp.int32)
    return jnp.max(plsc.load_gather(cnt_vm, [jv, z]))


def _sc_body(batch_hbm, z_hbm, q_hbm, posT_hbm, cnt_hbm,
             mass_hbm, out_hbm,
             cnt_vm, mass_vm, bufs, acc_vm, cmx_vm, cmy_vm, cmz_vm, mq_vm,
             out_vm, sems):
    c = lax.axis_index("c")
    s = lax.axis_index("s")
    w = s * 2 + c
    segbase = w * SEG_PER

    pltpu.sync_copy(cnt_hbm, cnt_vm)
    pltpu.sync_copy(mass_hbm, mass_vm)
    start = _bound(cnt_vm, w)
    end = _bound(cnt_vm, w + 1)

    lane = lax.iota(jnp.int32, 16)
    zero16i = jnp.zeros((16,), jnp.int32)
    one16i = jnp.full((16,), 1, jnp.int32)
    two16i = jnp.full((16,), 2, jnp.int32)
    zero16f = jnp.zeros((16,), jnp.float32)
    one16f = jnp.ones((16,), jnp.float32)

    for k in range(SEG_PER * 8 // 16):
        acc_vm[pl.ds(k * 16, 16)] = zero16f
    for k in range(SEG_PER // 16):
        out_vm[pl.ds(k * 16, 16)] = zero16f

    t0 = start // CH
    t1 = (end + CH - 1) // CH
    nch = t1 - t0
    npairs = (nch + 1) // 2

    def issue(t, slot):
        base = t * CH
        sem = sems.at[slot]
        bat_vm, z_vm, q_vm, px_vm, py_vm, pz_vm = bufs[slot]
        pltpu.async_copy(batch_hbm.at[pl.ds(base, CH)], bat_vm, sem)
        pltpu.async_copy(z_hbm.at[pl.ds(base, CH)], z_vm, sem)
        pltpu.async_copy(q_hbm.at[pl.ds(base, CH)], q_vm, sem)
        pltpu.async_copy(posT_hbm.at[0, pl.ds(base, CH)], px_vm, sem)
        pltpu.async_copy(posT_hbm.at[1, pl.ds(base, CH)], py_vm, sem)
        pltpu.async_copy(posT_hbm.at[2, pl.ds(base, CH)], pz_vm, sem)

    def drain(slot):
        sem = sems.at[slot]
        bat_vm, z_vm, q_vm, px_vm, py_vm, pz_vm = bufs[slot]
        pltpu.make_async_copy(batch_hbm.at[pl.ds(0, CH)], bat_vm, sem).wait()
        pltpu.make_async_copy(z_hbm.at[pl.ds(0, CH)], z_vm, sem).wait()
        pltpu.make_async_copy(q_hbm.at[pl.ds(0, CH)], q_vm, sem).wait()
        pltpu.make_async_copy(posT_hbm.at[0, pl.ds(0, CH)], px_vm, sem).wait()
        pltpu.make_async_copy(posT_hbm.at[1, pl.ds(0, CH)], py_vm, sem).wait()
        pltpu.make_async_copy(posT_hbm.at[2, pl.ds(0, CH)], pz_vm, sem).wait()

    def load_vregs(base, k, slot):
        bat_vm, z_vm, q_vm, px_vm, py_vm, pz_vm = bufs[slot]
        off = k * 16
        b16 = bat_vm[pl.ds(off, 16)]
        z16 = z_vm[pl.ds(off, 16)]
        q16 = q_vm[pl.ds(off, 16)]
        px = px_vm[pl.ds(off, 16)]
        py = py_vm[pl.ds(off, 16)]
        pz = pz_vm[pl.ds(off, 16)]
        aidx = base + off + lane
        msk = (aidx >= start) & (aidx < end)
        rel = jnp.clip(b16 - segbase, 0, SEG_PER - 1)
        return z16, q16, px, py, pz, msk, rel

    def pass1_vreg(base, k, slot):
        z16, q16, px, py, pz, msk, rel = load_vregs(base, k, slot)
        m16 = plsc.load_gather(mass_vm, [z16])
        i8 = rel * 8
        plsc.addupdate_scatter(acc_vm, [i8], m16, mask=msk)
        plsc.addupdate_scatter(acc_vm, [i8 + 1], m16 * px, mask=msk)
        plsc.addupdate_scatter(acc_vm, [i8 + 2], m16 * py, mask=msk)
        plsc.addupdate_scatter(acc_vm, [i8 + 3], m16 * pz, mask=msk)
        plsc.addupdate_scatter(acc_vm, [i8 + 4], q16, mask=msk)
        plsc.addupdate_scatter(acc_vm, [i8 + 5], one16f, mask=msk)

    def pass2_vreg(base, k, slot):
        z16, q16, px, py, pz, msk, rel = load_vregs(base, k, slot)
        cmx = plsc.load_gather(cmx_vm, [rel])
        cmy = plsc.load_gather(cmy_vm, [rel])
        cmz = plsc.load_gather(cmz_vm, [rel])
        mq = plsc.load_gather(mq_vm, [rel])
        dx = (px - cmx) * A_TO_A0
        dy = (py - cmy) * A_TO_A0
        dz = (pz - cmz) * A_TO_A0
        r2 = dx * dx + dy * dy + dz * dz
        cloud = jnp.abs(q16 - mq - z16.astype(jnp.float32))
        plsc.addupdate_scatter(out_vm, [rel], cloud * r2, mask=msk)

    def run_pass(vreg_fn):
        @pl.when(nch > 0)
        def _():
            issue(t0, 0)

        def pair_body(j, carry):
            t_a = t0 + 2 * j
            t_b = t_a + 1

            @pl.when(t_b < t1)
            def _():
                issue(t_b, 1)

            drain(0)

            def inner_a(k, c2):
                vreg_fn(t_a * CH, k, 0)
                return c2

            lax.fori_loop(0, CH // 16, inner_a, 0, unroll=4)

            @pl.when(t_b + 1 < t1)
            def _():
                issue(t_b + 1, 0)

            @pl.when(t_b < t1)
            def _():
                drain(1)

                def inner_b(k, c2):
                    vreg_fn(t_b * CH, k, 1)
                    return c2

                lax.fori_loop(0, CH // 16, inner_b, 0, unroll=4)

            return carry

        lax.fori_loop(0, npairs, pair_body, 0)

    run_pass(pass1_vreg)

    for k in range(SEG_PER // 16):
        sidx = (k * 16 + lane) * 8
        sm = plsc.load_gather(acc_vm, [sidx])
        smx = plsc.load_gather(acc_vm, [sidx + 1])
        smy = plsc.load_gather(acc_vm, [sidx + 2])
        smz = plsc.load_gather(acc_vm, [sidx + 3])
        sq = plsc.load_gather(acc_vm, [sidx + 4])
        n = plsc.load_gather(acc_vm, [sidx + 5])
        cmx_vm[pl.ds(k * 16, 16)] = smx / sm
        cmy_vm[pl.ds(k * 16, 16)] = smy / sm
        cmz_vm[pl.ds(k * 16, 16)] = smz / sm
        mq_vm[pl.ds(k * 16, 16)] = sq / n

    run_pass(pass2_vreg)

    pltpu.sync_copy(out_vm, out_hbm.at[pl.ds(segbase, SEG_PER)])


_sc_call = functools.partial(
    pl.kernel,
    out_type=jax.ShapeDtypeStruct((B,), jnp.float32),
    mesh=plsc.VectorSubcoreMesh(core_axis_name="c", subcore_axis_name="s"),
    scratch_types=[
        pltpu.VMEM((64, 8), jnp.int32),
        pltpu.VMEM((16,), jnp.float32),
        [[pltpu.VMEM((CH,), jnp.int32),
          pltpu.VMEM((CH,), jnp.int32),
          pltpu.VMEM((CH,), jnp.float32),
          pltpu.VMEM((CH,), jnp.float32),
          pltpu.VMEM((CH,), jnp.float32),
          pltpu.VMEM((CH,), jnp.float32)]
         for _ in range(2)],
        pltpu.VMEM((SEG_PER * 8,), jnp.float32),
        pltpu.VMEM((SEG_PER,), jnp.float32),
        pltpu.VMEM((SEG_PER,), jnp.float32),
        pltpu.VMEM((SEG_PER,), jnp.float32),
        pltpu.VMEM((SEG_PER,), jnp.float32),
        pltpu.VMEM((SEG_PER,), jnp.float32),
        pltpu.SemaphoreType.DMA((2,)),
    ],
    compiler_params=pltpu.CompilerParams(needs_layout_passes=False,
                                         use_tc_tiling_on_sc=False),
)(_sc_body)


def kernel(x, pos, Z, batch, W0, b0, W1, b1):
    batchp = jnp.pad(batch.astype(jnp.int32), (0, NP - N),
                     constant_values=PAD_SEG)
    zp = jnp.pad(Z.reshape(N).astype(jnp.int32), (0, NP - N))
    posT = jnp.pad(pos.T, ((0, 0), (0, NP - N)))
    q1, cnt = _mlp_call(x, W0, b0.reshape(1, H), W1, b1.reshape(1, 1), batchp)
    out = _sc_call(batchp, zp, q1, posT, cnt, _MASS16)
    return out.reshape(B, 1)


# speedup vs baseline: 2.3619x; 1.1925x over previous
import functools

import jax
import jax.numpy as jnp
from jax import lax
from jax.experimental import pallas as pl
from jax.experimental.pallas import tpu as pltpu
from jax.experimental.pallas import tpu_sc as plsc

N = 320000
B = 4096
D = 128
H = 64

MEAN = 0.7546106515883616
STD = 0.30338715545464656
A_TO_A0 = 1.8897268777743552

NSC = 32
SEG_PER = B // NSC

BLKA = 4096
NBLK = -(-N // BLKA)
NP = NBLK * BLKA
PAD_SEG = 2 * B

CH = 2048

_MASS16 = jnp.array(
    [0.0, 1.00784, 0.0, 0.0, 0.0, 0.0, 12.0107, 14.0067, 15.999, 18.998403,
     0.0, 0.0, 0.0, 0.0, 0.0, 0.0], dtype=jnp.float32)

_SEL43 = jnp.concatenate(
    [jnp.eye(3, dtype=jnp.float32), jnp.zeros((1, 3), jnp.float32)], axis=0)


def _mlp_body(x_ref, w0_ref, b0_ref, w1_ref, b1_ref, batch_ref, q_ref,
              cnt_ref):
    i = pl.program_id(0)
    x = x_ref[...]
    h = lax.dot_general(x, w0_ref[...], (((1,), (1,)), ((), ())),
                        preferred_element_type=jnp.float32)
    h = h + b0_ref[...]
    h = h * jax.nn.sigmoid(h)
    q = lax.dot_general(w1_ref[...], h, (((1,), (1,)), ((), ())),
                        preferred_element_type=jnp.float32)
    q = (q + b1_ref[...]) * STD + MEAN
    q_ref[...] = q.reshape(BLKA)

    bb = batch_ref[...]
    th = lax.broadcasted_iota(jnp.int32, (64, BLKA), 0) * SEG_PER
    cmp = (bb[None, :] < th).astype(jnp.int32)
    partial = jnp.sum(cmp, axis=1, keepdims=True)

    @pl.when(i == 0)
    def _():
        cnt_ref[...] = jnp.zeros_like(cnt_ref)

    cnt_ref[...] += jnp.broadcast_to(partial, (64, 8))


def _mlp_call(x, W0, b0_2d, W1, b1_2d, batchp):
    vec = pl.BlockSpec((BLKA,), lambda i: (i,))
    return pl.pallas_call(
        _mlp_body,
        grid=(NBLK,),
        in_specs=[
            pl.BlockSpec((BLKA, D), lambda i: (i, 0)),
            pl.BlockSpec((H, D), lambda i: (0, 0)),
            pl.BlockSpec((1, H), lambda i: (0, 0)),
            pl.BlockSpec((1, H), lambda i: (0, 0)),
            pl.BlockSpec((1, 1), lambda i: (0, 0)),
            vec,
        ],
        out_specs=[vec, pl.BlockSpec((64, 8), lambda i: (0, 0))],
        out_shape=[
            jax.ShapeDtypeStruct((NP,), jnp.float32),
            jax.ShapeDtypeStruct((64, 8), jnp.int32),
        ],
    )(x, W0, b0_2d, W1, b1_2d, batchp)


def _bound(cnt_vm, j):
    jv = jnp.full((16,), j, jnp.int32)
    z = jnp.zeros((16,), jnp.int32)
    return jnp.max(plsc.load_gather(cnt_vm, [jv, z]))


def _sc_body(batch_hbm, z_hbm, q_hbm, posT_hbm, cnt_hbm,
             mass_hbm, out_hbm,
             cnt_vm, mass_vm, bufs, acc_vm, cmx_vm, cmy_vm, cmz_vm, mq_vm,
             out_vm, sems):
    c = lax.axis_index("c")
    s = lax.axis_index("s")
    w = s * 2 + c
    segbase = w * SEG_PER

    pltpu.sync_copy(cnt_hbm, cnt_vm)
    pltpu.sync_copy(mass_hbm, mass_vm)
    start = _bound(cnt_vm, w)
    end = _bound(cnt_vm, w + 1)

    lane = lax.iota(jnp.int32, 16)
    zero16i = jnp.zeros((16,), jnp.int32)
    one16i = jnp.full((16,), 1, jnp.int32)
    two16i = jnp.full((16,), 2, jnp.int32)
    zero16f = jnp.zeros((16,), jnp.float32)
    one16f = jnp.ones((16,), jnp.float32)

    for k in range(SEG_PER * 8 // 16):
        acc_vm[pl.ds(k * 16, 16)] = zero16f
    for k in range(SEG_PER // 16):
        out_vm[pl.ds(k * 16, 16)] = zero16f

    t0 = start // CH
    t1 = (end + CH - 1) // CH
    nch = t1 - t0
    npairs = (nch + 1) // 2

    def issue(t, slot):
        base = t * CH
        sem = sems.at[slot]
        bat_vm, z_vm, q_vm, px_vm, py_vm, pz_vm = bufs[slot]
        pltpu.async_copy(batch_hbm.at[pl.ds(base, CH)], bat_vm, sem)
        pltpu.async_copy(z_hbm.at[pl.ds(base, CH)], z_vm, sem)
        pltpu.async_copy(q_hbm.at[pl.ds(base, CH)], q_vm, sem)
        pltpu.async_copy(posT_hbm.at[0, pl.ds(base, CH)], px_vm, sem)
        pltpu.async_copy(posT_hbm.at[1, pl.ds(base, CH)], py_vm, sem)
        pltpu.async_copy(posT_hbm.at[2, pl.ds(base, CH)], pz_vm, sem)

    def drain(slot):
        sem = sems.at[slot]
        bat_vm, z_vm, q_vm, px_vm, py_vm, pz_vm = bufs[slot]
        pltpu.make_async_copy(batch_hbm.at[pl.ds(0, CH)], bat_vm, sem).wait()
        pltpu.make_async_copy(z_hbm.at[pl.ds(0, CH)], z_vm, sem).wait()
        pltpu.make_async_copy(q_hbm.at[pl.ds(0, CH)], q_vm, sem).wait()
        pltpu.make_async_copy(posT_hbm.at[0, pl.ds(0, CH)], px_vm, sem).wait()
        pltpu.make_async_copy(posT_hbm.at[1, pl.ds(0, CH)], py_vm, sem).wait()
        pltpu.make_async_copy(posT_hbm.at[2, pl.ds(0, CH)], pz_vm, sem).wait()

    def load_vregs(base, k, slot):
        bat_vm, z_vm, q_vm, px_vm, py_vm, pz_vm = bufs[slot]
        off = k * 16
        b16 = bat_vm[pl.ds(off, 16)]
        z16 = z_vm[pl.ds(off, 16)]
        q16 = q_vm[pl.ds(off, 16)]
        px = px_vm[pl.ds(off, 16)]
        py = py_vm[pl.ds(off, 16)]
        pz = pz_vm[pl.ds(off, 16)]
        aidx = base + off + lane
        msk = (aidx >= start) & (aidx < end)
        rel = jnp.clip(b16 - segbase, 0, SEG_PER - 1)
        return z16, q16, px, py, pz, msk, rel

    def pass1_vreg(base, k, slot):
        z16, q16, px, py, pz, msk, rel = load_vregs(base, k, slot)
        m16 = plsc.load_gather(mass_vm, [z16])
        i8 = rel * 8
        plsc.addupdate_scatter(acc_vm, [i8], m16, mask=msk)
        plsc.addupdate_scatter(acc_vm, [i8 + 1], m16 * px, mask=msk)
        plsc.addupdate_scatter(acc_vm, [i8 + 2], m16 * py, mask=msk)
        plsc.addupdate_scatter(acc_vm, [i8 + 3], m16 * pz, mask=msk)
        plsc.addupdate_scatter(acc_vm, [i8 + 4], q16, mask=msk)
        plsc.addupdate_scatter(acc_vm, [i8 + 5], one16f, mask=msk)

    def pass2_vreg(base, k, slot):
        z16, q16, px, py, pz, msk, rel = load_vregs(base, k, slot)
        cmx = plsc.load_gather(cmx_vm, [rel])
        cmy = plsc.load_gather(cmy_vm, [rel])
        cmz = plsc.load_gather(cmz_vm, [rel])
        mq = plsc.load_gather(mq_vm, [rel])
        dx = (px - cmx) * A_TO_A0
        dy = (py - cmy) * A_TO_A0
        dz = (pz - cmz) * A_TO_A0
        r2 = dx * dx + dy * dy + dz * dz
        cloud = jnp.abs(q16 - mq - z16.astype(jnp.float32))
        plsc.addupdate_scatter(out_vm, [rel], cloud * r2, mask=msk)

    def run_pass(vreg_fn):
        @pl.when(nch > 0)
        def _():
            issue(t0, 0)

        def pair_body(j, carry):
            t_a = t0 + 2 * j
            t_b = t_a + 1

            @pl.when(t_b < t1)
            def _():
                issue(t_b, 1)

            drain(0)

            def inner_a(k, c2):
                vreg_fn(t_a * CH, k, 0)
                return c2

            lax.fori_loop(0, CH // 16, inner_a, 0, unroll=4)

            @pl.when(t_b + 1 < t1)
            def _():
                issue(t_b + 1, 0)

            @pl.when(t_b < t1)
            def _():
                drain(1)

                def inner_b(k, c2):
                    vreg_fn(t_b * CH, k, 1)
                    return c2

                lax.fori_loop(0, CH // 16, inner_b, 0, unroll=4)

            return carry

        lax.fori_loop(0, npairs, pair_body, 0)

    run_pass(pass1_vreg)

    for k in range(SEG_PER // 16):
        sidx = (k * 16 + lane) * 8
        sm = plsc.load_gather(acc_vm, [sidx])
        smx = plsc.load_gather(acc_vm, [sidx + 1])
        smy = plsc.load_gather(acc_vm, [sidx + 2])
        smz = plsc.load_gather(acc_vm, [sidx + 3])
        sq = plsc.load_gather(acc_vm, [sidx + 4])
        n = plsc.load_gather(acc_vm, [sidx + 5])
        cmx_vm[pl.ds(k * 16, 16)] = smx / sm
        cmy_vm[pl.ds(k * 16, 16)] = smy / sm
        cmz_vm[pl.ds(k * 16, 16)] = smz / sm
        mq_vm[pl.ds(k * 16, 16)] = sq / n

    run_pass(pass2_vreg)

    pltpu.sync_copy(out_vm, out_hbm.at[pl.ds(segbase, SEG_PER)])


_sc_call = functools.partial(
    pl.kernel,
    out_type=jax.ShapeDtypeStruct((B,), jnp.float32),
    mesh=plsc.VectorSubcoreMesh(core_axis_name="c", subcore_axis_name="s"),
    scratch_types=[
        pltpu.VMEM((64, 8), jnp.int32),
        pltpu.VMEM((16,), jnp.float32),
        [[pltpu.VMEM((CH,), jnp.int32),
          pltpu.VMEM((CH,), jnp.int32),
          pltpu.VMEM((CH,), jnp.float32),
          pltpu.VMEM((CH,), jnp.float32),
          pltpu.VMEM((CH,), jnp.float32),
          pltpu.VMEM((CH,), jnp.float32)]
         for _ in range(2)],
        pltpu.VMEM((SEG_PER * 8,), jnp.float32),
        pltpu.VMEM((SEG_PER,), jnp.float32),
        pltpu.VMEM((SEG_PER,), jnp.float32),
        pltpu.VMEM((SEG_PER,), jnp.float32),
        pltpu.VMEM((SEG_PER,), jnp.float32),
        pltpu.VMEM((SEG_PER,), jnp.float32),
        pltpu.SemaphoreType.DMA((2,)),
    ],
    compiler_params=pltpu.CompilerParams(needs_layout_passes=False,
                                         use_tc_tiling_on_sc=False),
)(_sc_body)


def kernel(x, pos, Z, batch, W0, b0, W1, b1):
    batchp = jnp.pad(batch.astype(jnp.int32), (0, NP - N),
                     constant_values=PAD_SEG)
    zp = jnp.pad(Z.reshape(N).astype(jnp.int32), (0, NP - N))
    posT = jnp.pad(pos.T, ((0, 0), (0, NP - N)))
    q1, cnt = _mlp_call(x, W0, b0.reshape(1, H), W1, b1.reshape(1, 1), batchp)
    out = _sc_call(batchp, zp, q1, posT, cnt, _MASS16)
    return out.reshape(B, 1)


# speedup vs baseline: 2.6114x; 1.1056x over previous
import functools

import jax
import jax.numpy as jnp
from jax import lax
from jax.experimental import pallas as pl
from jax.experimental.pallas import tpu as pltpu
from jax.experimental.pallas import tpu_sc as plsc

N = 320000
B = 4096
D = 128
H = 64

MEAN = 0.7546106515883616
STD = 0.30338715545464656
A_TO_A0 = 1.8897268777743552

NSC = 32
SEG_PER = B // NSC

BLKA = 8192
NBLK = -(-N // BLKA)
NP = NBLK * BLKA
PAD_SEG = 2 * B

CH = 2048

_MASS16 = jnp.array(
    [0.0, 1.00784, 0.0, 0.0, 0.0, 0.0, 12.0107, 14.0067, 15.999, 18.998403,
     0.0, 0.0, 0.0, 0.0, 0.0, 0.0], dtype=jnp.float32)

_SEL43 = jnp.concatenate(
    [jnp.eye(3, dtype=jnp.float32), jnp.zeros((1, 3), jnp.float32)], axis=0)


def _mlp_body(x_ref, w0_ref, b0_ref, w1_ref, b1_ref, batch_ref, q_ref,
              cnt_ref):
    i = pl.program_id(0)
    x = x_ref[...]
    h = lax.dot_general(x, w0_ref[...], (((1,), (1,)), ((), ())),
                        preferred_element_type=jnp.float32)
    h = h + b0_ref[...]
    h = h * jax.nn.sigmoid(h)
    q = lax.dot_general(w1_ref[...], h, (((1,), (1,)), ((), ())),
                        preferred_element_type=jnp.float32)
    q = (q + b1_ref[...]) * STD + MEAN
    q_ref[...] = q.reshape(BLKA)

    bb = batch_ref[...]
    th = lax.broadcasted_iota(jnp.int32, (64, BLKA), 0) * SEG_PER
    cmp = (bb[None, :] < th).astype(jnp.int32)
    partial = jnp.sum(cmp, axis=1, keepdims=True)

    @pl.when(i == 0)
    def _():
        cnt_ref[...] = jnp.zeros_like(cnt_ref)

    cnt_ref[...] += jnp.broadcast_to(partial, (64, 8))


def _mlp_call(x, W0, b0_2d, W1, b1_2d, batchp):
    vec = pl.BlockSpec((BLKA,), lambda i: (i,))
    return pl.pallas_call(
        _mlp_body,
        grid=(NBLK,),
        in_specs=[
            pl.BlockSpec((BLKA, D), lambda i: (i, 0)),
            pl.BlockSpec((H, D), lambda i: (0, 0)),
            pl.BlockSpec((1, H), lambda i: (0, 0)),
            pl.BlockSpec((1, H), lambda i: (0, 0)),
            pl.BlockSpec((1, 1), lambda i: (0, 0)),
            vec,
        ],
        out_specs=[vec, pl.BlockSpec((64, 8), lambda i: (0, 0))],
        out_shape=[
            jax.ShapeDtypeStruct((NP,), jnp.float32),
            jax.ShapeDtypeStruct((64, 8), jnp.int32),
        ],
    )(x, W0, b0_2d, W1, b1_2d, batchp)


def _bound(cnt_vm, j):
    jv = jnp.full((16,), j, jnp.int32)
    z = jnp.zeros((16,), jnp.int32)
    return jnp.max(plsc.load_gather(cnt_vm, [jv, z]))


def _sc_body(batch_hbm, z_hbm, q_hbm, posT_hbm, cnt_hbm,
             mass_hbm, out_hbm,
             cnt_vm, mass_vm, bufs, acc_vm, cmx_vm, cmy_vm, cmz_vm, mq_vm,
             out_vm, sems):
    c = lax.axis_index("c")
    s = lax.axis_index("s")
    w = s * 2 + c
    segbase = w * SEG_PER

    pltpu.sync_copy(cnt_hbm, cnt_vm)
    pltpu.sync_copy(mass_hbm, mass_vm)
    start = _bound(cnt_vm, w)
    end = _bound(cnt_vm, w + 1)

    lane = lax.iota(jnp.int32, 16)
    zero16i = jnp.zeros((16,), jnp.int32)
    one16i = jnp.full((16,), 1, jnp.int32)
    two16i = jnp.full((16,), 2, jnp.int32)
    zero16f = jnp.zeros((16,), jnp.float32)
    one16f = jnp.ones((16,), jnp.float32)

    for k in range(SEG_PER * 8 // 16):
        acc_vm[pl.ds(k * 16, 16)] = zero16f
    for k in range(SEG_PER // 16):
        out_vm[pl.ds(k * 16, 16)] = zero16f

    t0 = start // CH
    t1 = (end + CH - 1) // CH
    nch = t1 - t0
    npairs = (nch + 1) // 2

    def issue(t, slot):
        base = t * CH
        sem = sems.at[slot]
        bat_vm, z_vm, q_vm, px_vm, py_vm, pz_vm = bufs[slot]
        pltpu.async_copy(batch_hbm.at[pl.ds(base, CH)], bat_vm, sem)
        pltpu.async_copy(z_hbm.at[pl.ds(base, CH)], z_vm, sem)
        pltpu.async_copy(q_hbm.at[pl.ds(base, CH)], q_vm, sem)
        pltpu.async_copy(posT_hbm.at[0, pl.ds(base, CH)], px_vm, sem)
        pltpu.async_copy(posT_hbm.at[1, pl.ds(base, CH)], py_vm, sem)
        pltpu.async_copy(posT_hbm.at[2, pl.ds(base, CH)], pz_vm, sem)

    def drain(slot):
        sem = sems.at[slot]
        bat_vm, z_vm, q_vm, px_vm, py_vm, pz_vm = bufs[slot]
        pltpu.make_async_copy(batch_hbm.at[pl.ds(0, CH)], bat_vm, sem).wait()
        pltpu.make_async_copy(z_hbm.at[pl.ds(0, CH)], z_vm, sem).wait()
        pltpu.make_async_copy(q_hbm.at[pl.ds(0, CH)], q_vm, sem).wait()
        pltpu.make_async_copy(posT_hbm.at[0, pl.ds(0, CH)], px_vm, sem).wait()
        pltpu.make_async_copy(posT_hbm.at[1, pl.ds(0, CH)], py_vm, sem).wait()
        pltpu.make_async_copy(posT_hbm.at[2, pl.ds(0, CH)], pz_vm, sem).wait()

    def load_vregs(base, k, slot):
        bat_vm, z_vm, q_vm, px_vm, py_vm, pz_vm = bufs[slot]
        off = k * 16
        b16 = bat_vm[pl.ds(off, 16)]
        z16 = z_vm[pl.ds(off, 16)]
        q16 = q_vm[pl.ds(off, 16)]
        px = px_vm[pl.ds(off, 16)]
        py = py_vm[pl.ds(off, 16)]
        pz = pz_vm[pl.ds(off, 16)]
        aidx = base + off + lane
        msk = (aidx >= start) & (aidx < end)
        rel = jnp.clip(b16 - segbase, 0, SEG_PER - 1)
        return z16, q16, px, py, pz, msk, rel

    def pass1_vreg(base, k, slot):
        z16, q16, px, py, pz, msk, rel = load_vregs(base, k, slot)
        m16 = plsc.load_gather(mass_vm, [z16])
        i8 = rel * 8
        plsc.addupdate_scatter(acc_vm, [i8], m16, mask=msk)
        plsc.addupdate_scatter(acc_vm, [i8 + 1], m16 * px, mask=msk)
        plsc.addupdate_scatter(acc_vm, [i8 + 2], m16 * py, mask=msk)
        plsc.addupdate_scatter(acc_vm, [i8 + 3], m16 * pz, mask=msk)
        plsc.addupdate_scatter(acc_vm, [i8 + 4], q16, mask=msk)
        plsc.addupdate_scatter(acc_vm, [i8 + 5], one16f, mask=msk)

    def pass2_vreg(base, k, slot):
        z16, q16, px, py, pz, msk, rel = load_vregs(base, k, slot)
        cmx = plsc.load_gather(cmx_vm, [rel])
        cmy = plsc.load_gather(cmy_vm, [rel])
        cmz = plsc.load_gather(cmz_vm, [rel])
        mq = plsc.load_gather(mq_vm, [rel])
        dx = (px - cmx) * A_TO_A0
        dy = (py - cmy) * A_TO_A0
        dz = (pz - cmz) * A_TO_A0
        r2 = dx * dx + dy * dy + dz * dz
        cloud = jnp.abs(q16 - mq - z16.astype(jnp.float32))
        plsc.addupdate_scatter(out_vm, [rel], cloud * r2, mask=msk)

    def run_pass(vreg_fn):
        @pl.when(nch > 0)
        def _():
            issue(t0, 0)

        def pair_body(j, carry):
            t_a = t0 + 2 * j
            t_b = t_a + 1

            @pl.when(t_b < t1)
            def _():
                issue(t_b, 1)

            drain(0)

            def inner_a(k, c2):
                vreg_fn(t_a * CH, k, 0)
                return c2

            lax.fori_loop(0, CH // 16, inner_a, 0, unroll=4)

            @pl.when(t_b + 1 < t1)
            def _():
                issue(t_b + 1, 0)

            @pl.when(t_b < t1)
            def _():
                drain(1)

                def inner_b(k, c2):
                    vreg_fn(t_b * CH, k, 1)
                    return c2

                lax.fori_loop(0, CH // 16, inner_b, 0, unroll=4)

            return carry

        lax.fori_loop(0, npairs, pair_body, 0)

    run_pass(pass1_vreg)

    for k in range(SEG_PER // 16):
        sidx = (k * 16 + lane) * 8
        sm = plsc.load_gather(acc_vm, [sidx])
        smx = plsc.load_gather(acc_vm, [sidx + 1])
        smy = plsc.load_gather(acc_vm, [sidx + 2])
        smz = plsc.load_gather(acc_vm, [sidx + 3])
        sq = plsc.load_gather(acc_vm, [sidx + 4])
        n = plsc.load_gather(acc_vm, [sidx + 5])
        cmx_vm[pl.ds(k * 16, 16)] = smx / sm
        cmy_vm[pl.ds(k * 16, 16)] = smy / sm
        cmz_vm[pl.ds(k * 16, 16)] = smz / sm
        mq_vm[pl.ds(k * 16, 16)] = sq / n

    run_pass(pass2_vreg)

    pltpu.sync_copy(out_vm, out_hbm.at[pl.ds(segbase, SEG_PER)])


_sc_call = functools.partial(
    pl.kernel,
    out_type=jax.ShapeDtypeStruct((B,), jnp.float32),
    mesh=plsc.VectorSubcoreMesh(core_axis_name="c", subcore_axis_name="s"),
    scratch_types=[
        pltpu.VMEM((64, 8), jnp.int32),
        pltpu.VMEM((16,), jnp.float32),
        [[pltpu.VMEM((CH,), jnp.int32),
          pltpu.VMEM((CH,), jnp.int32),
          pltpu.VMEM((CH,), jnp.float32),
          pltpu.VMEM((CH,), jnp.float32),
          pltpu.VMEM((CH,), jnp.float32),
          pltpu.VMEM((CH,), jnp.float32)]
         for _ in range(2)],
        pltpu.VMEM((SEG_PER * 8,), jnp.float32),
        pltpu.VMEM((SEG_PER,), jnp.float32),
        pltpu.VMEM((SEG_PER,), jnp.float32),
        pltpu.VMEM((SEG_PER,), jnp.float32),
        pltpu.VMEM((SEG_PER,), jnp.float32),
        pltpu.VMEM((SEG_PER,), jnp.float32),
        pltpu.SemaphoreType.DMA((2,)),
    ],
    compiler_params=pltpu.CompilerParams(needs_layout_passes=False,
                                         use_tc_tiling_on_sc=False),
)(_sc_body)


def kernel(x, pos, Z, batch, W0, b0, W1, b1):
    batchp = jnp.pad(batch.astype(jnp.int32), (0, NP - N),
                     constant_values=PAD_SEG)
    zp = jnp.pad(Z.reshape(N).astype(jnp.int32), (0, NP - N))
    posT = jnp.pad(pos.T, ((0, 0), (0, NP - N)))
    q1, cnt = _mlp_call(x, W0, b0.reshape(1, H), W1, b1.reshape(1, 1), batchp)
    out = _sc_call(batchp, zp, q1, posT, cnt, _MASS16)
    return out.reshape(B, 1)


# speedup vs baseline: 2.7708x; 1.0611x over previous
import functools

import jax
import jax.numpy as jnp
from jax import lax
from jax.experimental import pallas as pl
from jax.experimental.pallas import tpu as pltpu
from jax.experimental.pallas import tpu_sc as plsc

N = 320000
B = 4096
D = 128
H = 64

MEAN = 0.7546106515883616
STD = 0.30338715545464656
A_TO_A0 = 1.8897268777743552

NSC = 32
SEG_PER = B // NSC

BLKA = 16384
NBLK = -(-N // BLKA)
NP = NBLK * BLKA
PAD_SEG = 2 * B

CH = 2048

_MASS16 = jnp.array(
    [0.0, 1.00784, 0.0, 0.0, 0.0, 0.0, 12.0107, 14.0067, 15.999, 18.998403,
     0.0, 0.0, 0.0, 0.0, 0.0, 0.0], dtype=jnp.float32)

_SEL43 = jnp.concatenate(
    [jnp.eye(3, dtype=jnp.float32), jnp.zeros((1, 3), jnp.float32)], axis=0)


def _mlp_body(x_ref, w0_ref, b0_ref, w1_ref, b1_ref, batch_ref, q_ref,
              cnt_ref):
    i = pl.program_id(0)
    x = x_ref[...]
    h = lax.dot_general(x, w0_ref[...], (((1,), (1,)), ((), ())),
                        preferred_element_type=jnp.float32)
    h = h + b0_ref[...]
    h = h * jax.nn.sigmoid(h)
    q = lax.dot_general(w1_ref[...], h, (((1,), (1,)), ((), ())),
                        preferred_element_type=jnp.float32)
    q = (q + b1_ref[...]) * STD + MEAN
    q_ref[...] = q.reshape(BLKA)

    bb = batch_ref[...]
    th = lax.broadcasted_iota(jnp.int32, (64, BLKA), 0) * SEG_PER
    cmp = (bb[None, :] < th).astype(jnp.int32)
    partial = jnp.sum(cmp, axis=1, keepdims=True)

    @pl.when(i == 0)
    def _():
        cnt_ref[...] = jnp.zeros_like(cnt_ref)

    cnt_ref[...] += jnp.broadcast_to(partial, (64, 8))


def _mlp_call(x, W0, b0_2d, W1, b1_2d, batchp):
    vec = pl.BlockSpec((BLKA,), lambda i: (i,))
    return pl.pallas_call(
        _mlp_body,
        grid=(NBLK,),
        in_specs=[
            pl.BlockSpec((BLKA, D), lambda i: (i, 0)),
            pl.BlockSpec((H, D), lambda i: (0, 0)),
            pl.BlockSpec((1, H), lambda i: (0, 0)),
            pl.BlockSpec((1, H), lambda i: (0, 0)),
            pl.BlockSpec((1, 1), lambda i: (0, 0)),
            vec,
        ],
        out_specs=[vec, pl.BlockSpec((64, 8), lambda i: (0, 0))],
        out_shape=[
            jax.ShapeDtypeStruct((NP,), jnp.float32),
            jax.ShapeDtypeStruct((64, 8), jnp.int32),
        ],
    )(x, W0, b0_2d, W1, b1_2d, batchp)


def _bound(cnt_vm, j):
    jv = jnp.full((16,), j, jnp.int32)
    z = jnp.zeros((16,), jnp.int32)
    return jnp.max(plsc.load_gather(cnt_vm, [jv, z]))


def _sc_body(batch_hbm, z_hbm, q_hbm, posT_hbm, cnt_hbm,
             mass_hbm, out_hbm,
             cnt_vm, mass_vm, bufs, acc_vm, cmx_vm, cmy_vm, cmz_vm, mq_vm,
             out_vm, sems):
    c = lax.axis_index("c")
    s = lax.axis_index("s")
    w = s * 2 + c
    segbase = w * SEG_PER

    pltpu.sync_copy(cnt_hbm, cnt_vm)
    pltpu.sync_copy(mass_hbm, mass_vm)
    start = _bound(cnt_vm, w)
    end = _bound(cnt_vm, w + 1)

    lane = lax.iota(jnp.int32, 16)
    zero16i = jnp.zeros((16,), jnp.int32)
    one16i = jnp.full((16,), 1, jnp.int32)
    two16i = jnp.full((16,), 2, jnp.int32)
    zero16f = jnp.zeros((16,), jnp.float32)
    one16f = jnp.ones((16,), jnp.float32)

    for k in range(SEG_PER * 8 // 16):
        acc_vm[pl.ds(k * 16, 16)] = zero16f
    for k in range(SEG_PER // 16):
        out_vm[pl.ds(k * 16, 16)] = zero16f

    t0 = start // CH
    t1 = (end + CH - 1) // CH
    nch = t1 - t0
    npairs = (nch + 1) // 2

    def issue(t, slot):
        base = t * CH
        sem = sems.at[slot]
        bat_vm, z_vm, q_vm, px_vm, py_vm, pz_vm = bufs[slot]
        pltpu.async_copy(batch_hbm.at[pl.ds(base, CH)], bat_vm, sem)
        pltpu.async_copy(z_hbm.at[pl.ds(base, CH)], z_vm, sem)
        pltpu.async_copy(q_hbm.at[pl.ds(base, CH)], q_vm, sem)
        pltpu.async_copy(posT_hbm.at[0, pl.ds(base, CH)], px_vm, sem)
        pltpu.async_copy(posT_hbm.at[1, pl.ds(base, CH)], py_vm, sem)
        pltpu.async_copy(posT_hbm.at[2, pl.ds(base, CH)], pz_vm, sem)

    def drain(slot):
        sem = sems.at[slot]
        bat_vm, z_vm, q_vm, px_vm, py_vm, pz_vm = bufs[slot]
        pltpu.make_async_copy(batch_hbm.at[pl.ds(0, CH)], bat_vm, sem).wait()
        pltpu.make_async_copy(z_hbm.at[pl.ds(0, CH)], z_vm, sem).wait()
        pltpu.make_async_copy(q_hbm.at[pl.ds(0, CH)], q_vm, sem).wait()
        pltpu.make_async_copy(posT_hbm.at[0, pl.ds(0, CH)], px_vm, sem).wait()
        pltpu.make_async_copy(posT_hbm.at[1, pl.ds(0, CH)], py_vm, sem).wait()
        pltpu.make_async_copy(posT_hbm.at[2, pl.ds(0, CH)], pz_vm, sem).wait()

    def load_vregs(base, k, slot):
        bat_vm, z_vm, q_vm, px_vm, py_vm, pz_vm = bufs[slot]
        off = k * 16
        b16 = bat_vm[pl.ds(off, 16)]
        z16 = z_vm[pl.ds(off, 16)]
        q16 = q_vm[pl.ds(off, 16)]
        px = px_vm[pl.ds(off, 16)]
        py = py_vm[pl.ds(off, 16)]
        pz = pz_vm[pl.ds(off, 16)]
        aidx = base + off + lane
        msk = (aidx >= start) & (aidx < end)
        rel = jnp.clip(b16 - segbase, 0, SEG_PER - 1)
        return z16, q16, px, py, pz, msk, rel

    def pass1_vreg(base, k, slot):
        z16, q16, px, py, pz, msk, rel = load_vregs(base, k, slot)
        m16 = plsc.load_gather(mass_vm, [z16])
        i8 = rel * 8
        plsc.addupdate_scatter(acc_vm, [i8], m16, mask=msk)
        plsc.addupdate_scatter(acc_vm, [i8 + 1], m16 * px, mask=msk)
        plsc.addupdate_scatter(acc_vm, [i8 + 2], m16 * py, mask=msk)
        plsc.addupdate_scatter(acc_vm, [i8 + 3], m16 * pz, mask=msk)
        plsc.addupdate_scatter(acc_vm, [i8 + 4], q16, mask=msk)
        plsc.addupdate_scatter(acc_vm, [i8 + 5], one16f, mask=msk)

    def pass2_vreg(base, k, slot):
        z16, q16, px, py, pz, msk, rel = load_vregs(base, k, slot)
        cmx = plsc.load_gather(cmx_vm, [rel])
        cmy = plsc.load_gather(cmy_vm, [rel])
        cmz = plsc.load_gather(cmz_vm, [rel])
        mq = plsc.load_gather(mq_vm, [rel])
        dx = (px - cmx) * A_TO_A0
        dy = (py - cmy) * A_TO_A0
        dz = (pz - cmz) * A_TO_A0
        r2 = dx * dx + dy * dy + dz * dz
        cloud = jnp.abs(q16 - mq - z16.astype(jnp.float32))
        plsc.addupdate_scatter(out_vm, [rel], cloud * r2, mask=msk)

    def run_pass(vreg_fn):
        @pl.when(nch > 0)
        def _():
            issue(t0, 0)

        def pair_body(j, carry):
            t_a = t0 + 2 * j
            t_b = t_a + 1

            @pl.when(t_b < t1)
            def _():
                issue(t_b, 1)

            drain(0)

            def inner_a(k, c2):
                vreg_fn(t_a * CH, k, 0)
                return c2

            lax.fori_loop(0, CH // 16, inner_a, 0, unroll=4)

            @pl.when(t_b + 1 < t1)
            def _():
                issue(t_b + 1, 0)

            @pl.when(t_b < t1)
            def _():
                drain(1)

                def inner_b(k, c2):
                    vreg_fn(t_b * CH, k, 1)
                    return c2

                lax.fori_loop(0, CH // 16, inner_b, 0, unroll=4)

            return carry

        lax.fori_loop(0, npairs, pair_body, 0)

    run_pass(pass1_vreg)

    for k in range(SEG_PER // 16):
        sidx = (k * 16 + lane) * 8
        sm = plsc.load_gather(acc_vm, [sidx])
        smx = plsc.load_gather(acc_vm, [sidx + 1])
        smy = plsc.load_gather(acc_vm, [sidx + 2])
        smz = plsc.load_gather(acc_vm, [sidx + 3])
        sq = plsc.load_gather(acc_vm, [sidx + 4])
        n = plsc.load_gather(acc_vm, [sidx + 5])
        cmx_vm[pl.ds(k * 16, 16)] = smx / sm
        cmy_vm[pl.ds(k * 16, 16)] = smy / sm
        cmz_vm[pl.ds(k * 16, 16)] = smz / sm
        mq_vm[pl.ds(k * 16, 16)] = sq / n

    run_pass(pass2_vreg)

    pltpu.sync_copy(out_vm, out_hbm.at[pl.ds(segbase, SEG_PER)])


_sc_call = functools.partial(
    pl.kernel,
    out_type=jax.ShapeDtypeStruct((B,), jnp.float32),
    mesh=plsc.VectorSubcoreMesh(core_axis_name="c", subcore_axis_name="s"),
    scratch_types=[
        pltpu.VMEM((64, 8), jnp.int32),
        pltpu.VMEM((16,), jnp.float32),
        [[pltpu.VMEM((CH,), jnp.int32),
          pltpu.VMEM((CH,), jnp.int32),
          pltpu.VMEM((CH,), jnp.float32),
          pltpu.VMEM((CH,), jnp.float32),
          pltpu.VMEM((CH,), jnp.float32),
          pltpu.VMEM((CH,), jnp.float32)]
         for _ in range(2)],
        pltpu.VMEM((SEG_PER * 8,), jnp.float32),
        pltpu.VMEM((SEG_PER,), jnp.float32),
        pltpu.VMEM((SEG_PER,), jnp.float32),
        pltpu.VMEM((SEG_PER,), jnp.float32),
        pltpu.VMEM((SEG_PER,), jnp.float32),
        pltpu.VMEM((SEG_PER,), jnp.float32),
        pltpu.SemaphoreType.DMA((2,)),
    ],
    compiler_params=pltpu.CompilerParams(needs_layout_passes=False,
                                         use_tc_tiling_on_sc=False),
)(_sc_body)


def kernel(x, pos, Z, batch, W0, b0, W1, b1):
    batchp = jnp.pad(batch.astype(jnp.int32), (0, NP - N),
                     constant_values=PAD_SEG)
    zp = jnp.pad(Z.reshape(N).astype(jnp.int32), (0, NP - N))
    posT = jnp.pad(pos.T, ((0, 0), (0, NP - N)))
    q1, cnt = _mlp_call(x, W0, b0.reshape(1, H), W1, b1.reshape(1, 1), batchp)
    out = _sc_call(batchp, zp, q1, posT, cnt, _MASS16)
    return out.reshape(B, 1)
